# baseline (speedup 1.0000x reference)
import functools

import jax
import jax.numpy as jnp
from jax import lax
from jax.experimental import pallas as pl
from jax.experimental.pallas import tpu as pltpu
from jax.experimental.pallas import tpu_sc as plsc

N_NODES = 10000
N_EDGES = 320000
EMBED = 128
HEADS = 8
HEAD_DIM = EMBED // HEADS

NC = 2
NS = 16
NW = NC * NS
CHUNK = 64
NCHUNK = 157
EPW = NCHUNK * CHUNK
N_EDGES_PAD = NW * EPW
N_NODES_PAD = 10016
N_DUMMY_EDGES = N_EDGES_PAD - N_EDGES
ROWS_PER_TILE = 624
ROWS_REMAINDER = N_NODES_PAD - NS * ROWS_PER_TILE

_DN_RHS_T = (((1,), (1,)), ((), ()))
_ROW_BLOCK = 1000
_GRID = N_NODES // _ROW_BLOCK


def _qkv_body(x_ref, wq_ref, wk_ref, wv_ref, q_ref, k_ref, v_ref):
    x = x_ref[...]
    q_ref[...] = lax.dot_general(x, wq_ref[...], _DN_RHS_T,
                                 preferred_element_type=jnp.float32)
    k_ref[...] = lax.dot_general(x, wk_ref[...], _DN_RHS_T,
                                 preferred_element_type=jnp.float32)
    v_ref[...] = lax.dot_general(x, wv_ref[...], _DN_RHS_T,
                                 preferred_element_type=jnp.float32)


def _qkv(x, wq, wk, wv):
    row_spec = pl.BlockSpec((_ROW_BLOCK, EMBED), lambda i: (i, 0))
    w_spec = pl.BlockSpec((EMBED, EMBED), lambda i: (0, 0))
    shape = jax.ShapeDtypeStruct((N_NODES, EMBED), jnp.float32)
    return pl.pallas_call(
        _qkv_body,
        grid=(_GRID,),
        in_specs=[row_spec, w_spec, w_spec, w_spec],
        out_specs=[row_spec, row_spec, row_spec],
        out_shape=[shape, shape, shape],
    )(x, wq, wk, wv)


def _edge_body(q_hbm, kv_hbm, src_hbm, dst_hbm, zeros_hbm,
               acc_out, z_out,
               srcq0, dstq0, srcq1, dstq1, srcv0, srcv1,
               qr0, kvr0, wr0, qr1, kvr1, wr1, zbuf, acc_sh,
               semg0, semg1, sems0, sems1, semi0, semi1):
    c = lax.axis_index("c")
    s = lax.axis_index("s")
    wid = s * NC + c
    row0 = pl.multiple_of(s * ROWS_PER_TILE, 8)
    ebase = pl.multiple_of(wid * EPW, 8)

    srcq = (srcq0, srcq1)
    dstq = (dstq0, dstq1)
    srcv = (srcv0, srcv1)
    qr = (qr0, qr1)
    kvr = (kvr0, kvr1)
    wr = (wr0, wr1)
    semg = (semg0, semg1)
    sems = (sems0, sems1)
    semi = (semi0, semi1)

    pltpu.sync_copy(zeros_hbm.at[pl.ds(row0, ROWS_PER_TILE)],
                    acc_sh.at[pl.ds(row0, ROWS_PER_TILE)])
    @pl.when(s == 0)
    def _zero_tail():
        pltpu.sync_copy(zeros_hbm.at[pl.ds(NS * ROWS_PER_TILE, ROWS_REMAINDER)],
                        acc_sh.at[pl.ds(NS * ROWS_PER_TILE, ROWS_REMAINDER)])
    plsc.subcore_barrier()

    for h in range(HEADS):
        zbuf[h] = jnp.zeros((16,), jnp.float32)

    lane = lax.iota(jnp.int32, 16)

    def idx_off(i):
        return pl.multiple_of(ebase + i * CHUNK, 8)

    def issue_idx(b, i):
        pltpu.async_copy(src_hbm.at[pl.ds(idx_off(i), CHUNK)], srcq[b], semi[b])
        pltpu.async_copy(dst_hbm.at[pl.ds(idx_off(i), CHUNK)], dstq[b], semi[b])

    def wait_idx(b, i):
        pltpu.make_async_copy(src_hbm.at[pl.ds(idx_off(i), CHUNK)],
                              srcq[b], semi[b]).wait()
        pltpu.make_async_copy(dst_hbm.at[pl.ds(idx_off(i), CHUNK)],
                              dstq[b], semi[b]).wait()

    def issue_gathers(b, i):
        pltpu.async_copy(src_hbm.at[pl.ds(idx_off(i), CHUNK)], srcv[b], semg[b])
        pltpu.async_copy(q_hbm.at[srcq[b]], qr[b], semg[b])
        pltpu.async_copy(kv_hbm.at[dstq[b]], kvr[b], semg[b])

    def wait_gathers(b):
        pltpu.make_async_copy(src_hbm.at[pl.ds(0, CHUNK)], srcv[b],
                              semg[b]).wait()
        pltpu.make_async_copy(q_hbm.at[srcq[b]], qr[b], semg[b]).wait()
        pltpu.make_async_copy(kv_hbm.at[dstq[b]], kvr[b], semg[b]).wait()

    def issue_scatter(b):
        pltpu.async_copy(wr[b], acc_sh.at[srcv[b]], sems[b], add=True)

    def wait_scatter(b):
        pltpu.make_async_copy(wr[b], acc_sh.at[srcv[b]], sems[b]).wait()

    def emit_group(qrb, kvrb, wrb, g):
        for h in range(HEADS):
            seg = pl.ds(h * HEAD_DIM, HEAD_DIM)
            svec = jnp.zeros((16,), jnp.float32)
            for e in range(16):
                erow = g * 16 + e
                kvb = plsc.bitcast(kvrb[erow, seg], jnp.bfloat16)
                kseg, vseg = plsc.unpack(kvb, format=plsc.PackFormat.INTERLEAVED)
                wrb[erow, seg] = vseg
                s_eh = jnp.sum(qrb[erow, seg] * kseg)
                svec = jnp.where(lane == e,
                                 lax.broadcast_in_dim(s_eh, (16,), ()),
                                 svec)
            p = jnp.exp(svec * 0.25)
            zbuf[h] = zbuf[h] + p
            for e in range(16):
                erow = g * 16 + e
                pv = lax.broadcast_in_dim(p[e], (HEAD_DIM,), ())
                wrb[erow, seg] = wrb[erow, seg] * pv

    def compute(b):
        def group_body(g, gcarry):
            emit_group(qr[b], kvr[b], wr[b], g)
            return gcarry
        lax.fori_loop(0, CHUNK // 16, group_body, 0)

    def step(i, b):
        @pl.when(i >= 1)
        def _drain_prev():
            wait_scatter(1 - b)
        @pl.when(i + 1 < NCHUNK)
        def _prefetch_next():
            wait_idx(1 - b, i + 1)
            issue_gathers(1 - b, i + 1)
        wait_gathers(b)
        @pl.when(i + 2 < NCHUNK)
        def _prefetch_idx():
            issue_idx(b, i + 2)
        compute(b)
        issue_scatter(b)

    pltpu.sync_copy(src_hbm.at[pl.ds(idx_off(0), CHUNK)], srcq[0])
    pltpu.sync_copy(dst_hbm.at[pl.ds(idx_off(0), CHUNK)], dstq[0])
    issue_gathers(0, 0)
    issue_idx(1, 1)

    def chunk_body(i, carry):
        @pl.when((i & 1) == 0)
        def _even():
            step(i, 0)
        @pl.when((i & 1) == 1)
        def _odd():
            step(i, 1)
        return carry

    lax.fori_loop(0, NCHUNK, chunk_body, 0)
    wait_scatter((NCHUNK - 1) & 1)

    plsc.subcore_barrier()
    pltpu.sync_copy(acc_sh.at[pl.ds(row0, ROWS_PER_TILE)],
                    acc_out.at[c, pl.ds(row0, ROWS_PER_TILE)])
    @pl.when(s == 0)
    def _copy_tail():
        pltpu.sync_copy(acc_sh.at[pl.ds(NS * ROWS_PER_TILE, ROWS_REMAINDER)],
                        acc_out.at[c, pl.ds(NS * ROWS_PER_TILE, ROWS_REMAINDER)])
    pltpu.sync_copy(zbuf, z_out.at[c, s])


_edge_kernel = functools.partial(
    pl.kernel,
    out_type=(
        jax.ShapeDtypeStruct((NC, N_NODES_PAD, EMBED), jnp.float32),
        jax.ShapeDtypeStruct((NC, NS, HEADS, 16), jnp.float32),
    ),
    mesh=plsc.VectorSubcoreMesh(core_axis_name="c", subcore_axis_name="s"),
    compiler_params=pltpu.CompilerParams(needs_layout_passes=False),
    scratch_types=[
        pltpu.VMEM((CHUNK,), jnp.int32),
        pltpu.VMEM((CHUNK,), jnp.int32),
        pltpu.VMEM((CHUNK,), jnp.int32),
        pltpu.VMEM((CHUNK,), jnp.int32),
        pltpu.VMEM((CHUNK,), jnp.int32),
        pltpu.VMEM((CHUNK,), jnp.int32),
        pltpu.VMEM((CHUNK, EMBED), jnp.float32),
        pltpu.VMEM((CHUNK, EMBED), jnp.float32),
        pltpu.VMEM((CHUNK, EMBED), jnp.float32),
        pltpu.VMEM((CHUNK, EMBED), jnp.float32),
        pltpu.VMEM((CHUNK, EMBED), jnp.float32),
        pltpu.VMEM((CHUNK, EMBED), jnp.float32),
        pltpu.VMEM((HEADS, 16), jnp.float32),
        pltpu.VMEM_SHARED((N_NODES_PAD, EMBED), jnp.float32),
        pltpu.SemaphoreType.DMA,
        pltpu.SemaphoreType.DMA,
        pltpu.SemaphoreType.DMA,
        pltpu.SemaphoreType.DMA,
        pltpu.SemaphoreType.DMA,
        pltpu.SemaphoreType.DMA,
    ],
)(_edge_body)


def _out_body(acc_ref, z_ref, wo_ref, bo_ref, o_ref):
    zs = jnp.sum(z_ref[...], axis=0, keepdims=True)
    seg_i = lax.broadcasted_iota(jnp.int32, (EMBED, EMBED), 0) // HEAD_DIM
    seg_j = lax.broadcasted_iota(jnp.int32, (EMBED, EMBED), 1) // HEAD_DIM
    seg = (seg_i == seg_j).astype(jnp.float32)
    zrow = lax.dot_general(zs, seg, (((1,), (0,)), ((), ())),
                           preferred_element_type=jnp.float32) - float(N_DUMMY_EDGES)
    a = (acc_ref[0] + acc_ref[1]) * (1.0 / zrow)
    o_ref[...] = lax.dot_general(a, wo_ref[...], _DN_RHS_T,
                                 preferred_element_type=jnp.float32) + bo_ref[...]


def _out_proj(acc2, zflat, wo, bo_row):
    return pl.pallas_call(
        _out_body,
        grid=(_GRID,),
        in_specs=[
            pl.BlockSpec((NC, _ROW_BLOCK, EMBED), lambda i: (0, i, 0)),
            pl.BlockSpec((NW, EMBED), lambda i: (0, 0)),
            pl.BlockSpec((EMBED, EMBED), lambda i: (0, 0)),
            pl.BlockSpec((1, EMBED), lambda i: (0, 0)),
        ],
        out_specs=pl.BlockSpec((_ROW_BLOCK, EMBED), lambda i: (i, 0)),
        out_shape=jax.ShapeDtypeStruct((N_NODES, EMBED), jnp.float32),
    )(acc2, zflat, wo, bo_row)


def kernel(embeddings, edge_index, Wq, Wk, Wv, Wo, bo):
    pad_ids = jnp.full((N_DUMMY_EDGES,), N_NODES, jnp.int32)
    src = jnp.concatenate([edge_index[0].astype(jnp.int32), pad_ids])
    dst = jnp.concatenate([edge_index[1].astype(jnp.int32), pad_ids])
    q, k, v = _qkv(embeddings, Wq, Wk, Wv)
    rowpad = jnp.zeros((N_NODES_PAD - N_NODES, EMBED), jnp.float32)
    q = jnp.concatenate([q, rowpad])
    k = jnp.concatenate([k, rowpad])
    v = jnp.concatenate([v, rowpad])
    kb = lax.bitcast_convert_type(k.astype(jnp.bfloat16),
                                  jnp.uint16).astype(jnp.uint32)
    vb = lax.bitcast_convert_type(v.astype(jnp.bfloat16),
                                  jnp.uint16).astype(jnp.uint32)
    kv = lax.bitcast_convert_type(kb | (vb << 16), jnp.float32)
    zeros = jnp.zeros((N_NODES_PAD, EMBED), jnp.float32)
    acc2, zpart = _edge_kernel(q, kv, src, dst, zeros)
    zflat = zpart.reshape(NW, EMBED)
    return _out_proj(acc2[:, :N_NODES, :], zflat, Wo, bo.reshape(1, EMBED))


# speedup vs baseline: 1.0229x; 1.0229x over previous
import functools

import jax
import jax.numpy as jnp
from jax import lax
from jax.experimental import pallas as pl
from jax.experimental.pallas import tpu as pltpu
from jax.experimental.pallas import tpu_sc as plsc

N_NODES = 10000
N_EDGES = 320000
EMBED = 128
HEADS = 8
HEAD_DIM = EMBED // HEADS

NC = 2
NS = 16
NW = NC * NS
CHUNK = 64
NCHUNK = 157
EPW = NCHUNK * CHUNK
N_EDGES_PAD = NW * EPW
N_NODES_PAD = 10016
N_DUMMY_EDGES = N_EDGES_PAD - N_EDGES
ROWS_PER_TILE = 624
ROWS_REMAINDER = N_NODES_PAD - NS * ROWS_PER_TILE

_DN_RHS_T = (((1,), (1,)), ((), ()))
_ROW_BLOCK = 1000
_GRID = N_NODES // _ROW_BLOCK


def _qkv_body(x_ref, wq_ref, wk_ref, wv_ref, q_ref, k_ref, v_ref):
    x = x_ref[...]
    q_ref[...] = lax.dot_general(x, wq_ref[...], _DN_RHS_T,
                                 preferred_element_type=jnp.float32)
    k_ref[...] = lax.dot_general(x, wk_ref[...], _DN_RHS_T,
                                 preferred_element_type=jnp.float32)
    v_ref[...] = lax.dot_general(x, wv_ref[...], _DN_RHS_T,
                                 preferred_element_type=jnp.float32)


def _qkv(x, wq, wk, wv):
    row_spec = pl.BlockSpec((_ROW_BLOCK, EMBED), lambda i: (i, 0))
    w_spec = pl.BlockSpec((EMBED, EMBED), lambda i: (0, 0))
    shape = jax.ShapeDtypeStruct((N_NODES, EMBED), jnp.float32)
    return pl.pallas_call(
        _qkv_body,
        grid=(_GRID,),
        in_specs=[row_spec, w_spec, w_spec, w_spec],
        out_specs=[row_spec, row_spec, row_spec],
        out_shape=[shape, shape, shape],
    )(x, wq, wk, wv)


def _edge_body(q_hbm, kv_hbm, src_hbm, dst_hbm, zeros_hbm,
               acc_out, z_out,
               srcq0, dstq0, srcq1, dstq1, srcv0, srcv1,
               qr0, kvr0, wr0, qr1, kvr1, wr1, zbuf, acc_sh,
               semg0, semg1, sems0, sems1, semi0, semi1):
    c = lax.axis_index("c")
    s = lax.axis_index("s")
    wid = s * NC + c
    row0 = pl.multiple_of(s * ROWS_PER_TILE, 8)
    ebase = pl.multiple_of(wid * EPW, 8)

    srcq = (srcq0, srcq1)
    dstq = (dstq0, dstq1)
    srcv = (srcv0, srcv1)
    qr = (qr0, qr1)
    kvr = (kvr0, kvr1)
    wr = (wr0, wr1)
    semg = (semg0, semg1)
    sems = (sems0, sems1)
    semi = (semi0, semi1)

    pltpu.sync_copy(zeros_hbm.at[pl.ds(row0, ROWS_PER_TILE)],
                    acc_sh.at[pl.ds(row0, ROWS_PER_TILE)])
    @pl.when(s == 0)
    def _zero_tail():
        pltpu.sync_copy(zeros_hbm.at[pl.ds(NS * ROWS_PER_TILE, ROWS_REMAINDER)],
                        acc_sh.at[pl.ds(NS * ROWS_PER_TILE, ROWS_REMAINDER)])
    plsc.subcore_barrier()

    for h in range(HEADS):
        zbuf[h] = jnp.zeros((16,), jnp.float32)

    lane = lax.iota(jnp.int32, 16)

    def idx_off(i):
        return pl.multiple_of(ebase + i * CHUNK, 8)

    def issue_idx(b, i):
        pltpu.async_copy(src_hbm.at[pl.ds(idx_off(i), CHUNK)], srcq[b], semi[b])
        pltpu.async_copy(dst_hbm.at[pl.ds(idx_off(i), CHUNK)], dstq[b], semi[b])

    def wait_idx(b, i):
        pltpu.make_async_copy(src_hbm.at[pl.ds(idx_off(i), CHUNK)],
                              srcq[b], semi[b]).wait()
        pltpu.make_async_copy(dst_hbm.at[pl.ds(idx_off(i), CHUNK)],
                              dstq[b], semi[b]).wait()

    def issue_gathers(b, i):
        pltpu.async_copy(src_hbm.at[pl.ds(idx_off(i), CHUNK)], srcv[b], semg[b])
        pltpu.async_copy(q_hbm.at[srcq[b]], qr[b], semg[b])
        pltpu.async_copy(kv_hbm.at[dstq[b]], kvr[b], semg[b])

    def wait_gathers(b):
        pltpu.make_async_copy(src_hbm.at[pl.ds(0, CHUNK)], srcv[b],
                              semg[b]).wait()
        pltpu.make_async_copy(q_hbm.at[srcq[b]], qr[b], semg[b]).wait()
        pltpu.make_async_copy(kv_hbm.at[dstq[b]], kvr[b], semg[b]).wait()

    def issue_scatter(b):
        pltpu.async_copy(wr[b], acc_sh.at[srcv[b]], sems[b], add=True)

    def wait_scatter(b):
        pltpu.make_async_copy(wr[b], acc_sh.at[srcv[b]], sems[b]).wait()

    def emit_group(qrb, kvrb, wrb, g):
        def kv_segs(erow, seg):
            kvb = plsc.bitcast(kvrb[erow, seg], jnp.bfloat16)
            return plsc.unpack(kvb, format=plsc.PackFormat.INTERLEAVED)

        for h in range(HEADS):
            seg = pl.ds(h * HEAD_DIM, HEAD_DIM)
            svec = jnp.zeros((16,), jnp.float32)
            for e in range(16):
                erow = g * 16 + e
                kseg, _ = kv_segs(erow, seg)
                s_eh = jnp.sum(qrb[erow, seg] * kseg)
                svec = jnp.where(lane == e,
                                 lax.broadcast_in_dim(s_eh, (16,), ()),
                                 svec)
            p = jnp.exp(svec * 0.25)
            zbuf[h] = zbuf[h] + p
            for e in range(16):
                erow = g * 16 + e
                _, vseg = kv_segs(erow, seg)
                pv = lax.broadcast_in_dim(p[e], (HEAD_DIM,), ())
                wrb[erow, seg] = vseg * pv

    def compute(b):
        def group_body(g, gcarry):
            emit_group(qr[b], kvr[b], wr[b], g)
            return gcarry
        lax.fori_loop(0, CHUNK // 16, group_body, 0)

    def step(i, b):
        @pl.when(i >= 1)
        def _drain_prev():
            wait_scatter(1 - b)
        @pl.when(i + 1 < NCHUNK)
        def _prefetch_next():
            wait_idx(1 - b, i + 1)
            issue_gathers(1 - b, i + 1)
        wait_gathers(b)
        @pl.when(i + 2 < NCHUNK)
        def _prefetch_idx():
            issue_idx(b, i + 2)
        compute(b)
        issue_scatter(b)

    pltpu.sync_copy(src_hbm.at[pl.ds(idx_off(0), CHUNK)], srcq[0])
    pltpu.sync_copy(dst_hbm.at[pl.ds(idx_off(0), CHUNK)], dstq[0])
    issue_gathers(0, 0)
    issue_idx(1, 1)

    def chunk_body(i, carry):
        @pl.when((i & 1) == 0)
        def _even():
            step(i, 0)
        @pl.when((i & 1) == 1)
        def _odd():
            step(i, 1)
        return carry

    lax.fori_loop(0, NCHUNK, chunk_body, 0)
    wait_scatter((NCHUNK - 1) & 1)

    plsc.subcore_barrier()
    pltpu.sync_copy(acc_sh.at[pl.ds(row0, ROWS_PER_TILE)],
                    acc_out.at[c, pl.ds(row0, ROWS_PER_TILE)])
    @pl.when(s == 0)
    def _copy_tail():
        pltpu.sync_copy(acc_sh.at[pl.ds(NS * ROWS_PER_TILE, ROWS_REMAINDER)],
                        acc_out.at[c, pl.ds(NS * ROWS_PER_TILE, ROWS_REMAINDER)])
    pltpu.sync_copy(zbuf, z_out.at[c, s])


_edge_kernel = functools.partial(
    pl.kernel,
    out_type=(
        jax.ShapeDtypeStruct((NC, N_NODES_PAD, EMBED), jnp.float32),
        jax.ShapeDtypeStruct((NC, NS, HEADS, 16), jnp.float32),
    ),
    mesh=plsc.VectorSubcoreMesh(core_axis_name="c", subcore_axis_name="s"),
    compiler_params=pltpu.CompilerParams(needs_layout_passes=False),
    scratch_types=[
        pltpu.VMEM((CHUNK,), jnp.int32),
        pltpu.VMEM((CHUNK,), jnp.int32),
        pltpu.VMEM((CHUNK,), jnp.int32),
        pltpu.VMEM((CHUNK,), jnp.int32),
        pltpu.VMEM((CHUNK,), jnp.int32),
        pltpu.VMEM((CHUNK,), jnp.int32),
        pltpu.VMEM((CHUNK, EMBED), jnp.float32),
        pltpu.VMEM((CHUNK, EMBED), jnp.float32),
        pltpu.VMEM((CHUNK, EMBED), jnp.float32),
        pltpu.VMEM((CHUNK, EMBED), jnp.float32),
        pltpu.VMEM((CHUNK, EMBED), jnp.float32),
        pltpu.VMEM((CHUNK, EMBED), jnp.float32),
        pltpu.VMEM((HEADS, 16), jnp.float32),
        pltpu.VMEM_SHARED((N_NODES_PAD, EMBED), jnp.float32),
        pltpu.SemaphoreType.DMA,
        pltpu.SemaphoreType.DMA,
        pltpu.SemaphoreType.DMA,
        pltpu.SemaphoreType.DMA,
        pltpu.SemaphoreType.DMA,
        pltpu.SemaphoreType.DMA,
    ],
)(_edge_body)


def _out_body(acc_ref, z_ref, wo_ref, bo_ref, o_ref):
    zs = jnp.sum(z_ref[...], axis=0, keepdims=True)
    seg_i = lax.broadcasted_iota(jnp.int32, (EMBED, EMBED), 0) // HEAD_DIM
    seg_j = lax.broadcasted_iota(jnp.int32, (EMBED, EMBED), 1) // HEAD_DIM
    seg = (seg_i == seg_j).astype(jnp.float32)
    zrow = lax.dot_general(zs, seg, (((1,), (0,)), ((), ())),
                           preferred_element_type=jnp.float32) - float(N_DUMMY_EDGES)
    a = (acc_ref[0] + acc_ref[1]) * (1.0 / zrow)
    o_ref[...] = lax.dot_general(a, wo_ref[...], _DN_RHS_T,
                                 preferred_element_type=jnp.float32) + bo_ref[...]


def _out_proj(acc2, zflat, wo, bo_row):
    return pl.pallas_call(
        _out_body,
        grid=(_GRID,),
        in_specs=[
            pl.BlockSpec((NC, _ROW_BLOCK, EMBED), lambda i: (0, i, 0)),
            pl.BlockSpec((NW, EMBED), lambda i: (0, 0)),
            pl.BlockSpec((EMBED, EMBED), lambda i: (0, 0)),
            pl.BlockSpec((1, EMBED), lambda i: (0, 0)),
        ],
        out_specs=pl.BlockSpec((_ROW_BLOCK, EMBED), lambda i: (i, 0)),
        out_shape=jax.ShapeDtypeStruct((N_NODES, EMBED), jnp.float32),
    )(acc2, zflat, wo, bo_row)


def kernel(embeddings, edge_index, Wq, Wk, Wv, Wo, bo):
    pad_ids = jnp.full((N_DUMMY_EDGES,), N_NODES, jnp.int32)
    src = jnp.concatenate([edge_index[0].astype(jnp.int32), pad_ids])
    dst = jnp.concatenate([edge_index[1].astype(jnp.int32), pad_ids])
    q, k, v = _qkv(embeddings, Wq, Wk, Wv)
    rowpad = jnp.zeros((N_NODES_PAD - N_NODES, EMBED), jnp.float32)
    q = jnp.concatenate([q, rowpad])
    k = jnp.concatenate([k, rowpad])
    v = jnp.concatenate([v, rowpad])
    kb = lax.bitcast_convert_type(k.astype(jnp.bfloat16),
                                  jnp.uint16).astype(jnp.uint32)
    vb = lax.bitcast_convert_type(v.astype(jnp.bfloat16),
                                  jnp.uint16).astype(jnp.uint32)
    kv = lax.bitcast_convert_type(kb | (vb << 16), jnp.float32)
    zeros = jnp.zeros((N_NODES_PAD, EMBED), jnp.float32)
    acc2, zpart = _edge_kernel(q, kv, src, dst, zeros)
    zflat = zpart.reshape(NW, EMBED)
    return _out_proj(acc2[:, :N_NODES, :], zflat, Wo, bo.reshape(1, EMBED))
